# baseline (speedup 1.0000x reference)
import functools

import jax
import jax.numpy as jnp
from jax import lax
from jax.experimental import pallas as pl
from jax.experimental.pallas import tpu as pltpu
from jax.experimental.pallas import tpu_sc as plsc

NC = 2
NS = 16
NW = NC * NS


@functools.lru_cache(maxsize=None)
def _build(B: int, F: int, V: int, D: int):
    assert B % NW == 0
    C = B // NW
    mesh = plsc.VectorSubcoreMesh(core_axis_name="c", subcore_axis_name="s")

    @functools.partial(
        pl.kernel,
        mesh=mesh,
        out_type=jax.ShapeDtypeStruct((B, F, D), jnp.float32),
        compiler_params=pltpu.CompilerParams(use_tc_tiling_on_sc=False),
        scratch_types=[
            pltpu.VMEM((C,), jnp.int32),
            pltpu.VMEM((C,), jnp.int32),
            pltpu.VMEM((C, D), jnp.float32),
            pltpu.VMEM((C, D), jnp.float32),
            pltpu.SemaphoreType.DMA,
            pltpu.SemaphoreType.DMA,
            pltpu.SemaphoreType.DMA,
            pltpu.SemaphoreType.DMA,
        ],
    )
    def gather_kernel(idx_hbm, table_hbm, out_hbm, idx_a, idx_b,
                      rows_a, rows_b, gsem0, gsem1, ssem0, ssem1):
        wid = lax.axis_index("s") * NC + lax.axis_index("c")
        b0 = wid * C
        idx_v = [idx_a, idx_b]
        rows_v = [rows_a, rows_b]
        gsem = [gsem0, gsem1]
        ssem = [ssem0, ssem1]
        gcp = [None, None]
        scp = [None, None]

        pltpu.sync_copy(idx_hbm.at[0, pl.ds(b0, C)], idx_a)
        gcp[0] = pltpu.async_copy(table_hbm.at[idx_a], rows_a, gsem[0])
        for f in range(F):
            buf = f % 2
            nbuf = (f + 1) % 2
            if f + 1 < F:
                if scp[nbuf] is not None:
                    scp[nbuf].wait()
                pltpu.sync_copy(idx_hbm.at[f + 1, pl.ds(b0, C)], idx_v[nbuf])
                gcp[nbuf] = pltpu.async_copy(
                    table_hbm.at[idx_v[nbuf]], rows_v[nbuf], gsem[nbuf])
            gcp[buf].wait()
            scp[buf] = pltpu.async_copy(
                rows_v[buf], out_hbm.at[pl.ds(b0, C), f], ssem[buf])
        for b in range(2):
            if scp[b] is not None:
                scp[b].wait()

    return gather_kernel


def kernel(global_indices, table):
    B, F = global_indices.shape
    V, D = table.shape
    idxT = global_indices.T.astype(jnp.int32)
    return _build(B, F, V, D)(idxT, table)


# speedup vs baseline: 1.4334x; 1.4334x over previous
import functools

import jax
import jax.numpy as jnp
from jax import lax
from jax.experimental import pallas as pl
from jax.experimental.pallas import tpu as pltpu
from jax.experimental.pallas import tpu_sc as plsc

NC = 2
NS = 16
NW = NC * NS

TBLK = 16384


def _transpose_body(x_ref, o_ref):
    y = x_ref[...].T
    q = TBLK // 4
    for u in range(4):
        o_ref[:, pl.ds(32 * u, 32)] = y[q * u:q * (u + 1), :]


@functools.lru_cache(maxsize=None)
def _build_transpose(V: int, D: int):
    grid = (V + TBLK - 1) // TBLK
    rows_blk = TBLK // 4
    return pl.pallas_call(
        _transpose_body,
        grid=(grid,),
        in_specs=[pl.BlockSpec((D, TBLK), lambda g: (0, g))],
        out_specs=pl.BlockSpec((rows_blk, 128), lambda g: (g, 0)),
        out_shape=jax.ShapeDtypeStruct((grid * rows_blk, 128),
                                       jnp.float32),
    )


@functools.lru_cache(maxsize=None)
def _build_gather(B: int, F: int, VP: int, D: int):
    assert B % NW == 0
    C = B // NW
    mesh = plsc.VectorSubcoreMesh(core_axis_name="c", subcore_axis_name="s")

    @functools.partial(
        pl.kernel,
        mesh=mesh,
        out_type=jax.ShapeDtypeStruct((B, F, D), jnp.float32),
        compiler_params=pltpu.CompilerParams(use_tc_tiling_on_sc=False),
        scratch_types=[
            pltpu.VMEM((C,), jnp.int32),
            pltpu.VMEM((C,), jnp.int32),
            pltpu.VMEM((C, D), jnp.float32),
            pltpu.VMEM((C, D), jnp.float32),
            pltpu.SemaphoreType.DMA,
            pltpu.SemaphoreType.DMA,
            pltpu.SemaphoreType.DMA,
            pltpu.SemaphoreType.DMA,
        ],
    )
    def gather_kernel(idx_hbm, table_hbm, out_hbm, idx_a, idx_b,
                      rows_a, rows_b, gsem0, gsem1, ssem0, ssem1):
        wid = lax.axis_index("s") * NC + lax.axis_index("c")
        b0 = wid * C
        idx_v = [idx_a, idx_b]
        rows_v = [rows_a, rows_b]
        gsem = [gsem0, gsem1]
        ssem = [ssem0, ssem1]
        gcp = [None, None]
        scp = [None, None]

        def permute(ref):
            for i in range(C // 16):
                v = ref[pl.ds(16 * i, 16)]
                m = ((v & -16384) | ((v & 4095) << 2)
                     | ((v >> 12) & 3))
                ref[pl.ds(16 * i, 16)] = m

        pltpu.sync_copy(idx_hbm.at[0, pl.ds(b0, C)], idx_a)
        permute(idx_a)
        gcp[0] = pltpu.async_copy(table_hbm.at[idx_a], rows_a, gsem[0])
        for f in range(F):
            buf = f % 2
            nbuf = (f + 1) % 2
            if f + 1 < F:
                if scp[nbuf] is not None:
                    scp[nbuf].wait()
                pltpu.sync_copy(idx_hbm.at[f + 1, pl.ds(b0, C)], idx_v[nbuf])
                permute(idx_v[nbuf])
                gcp[nbuf] = pltpu.async_copy(
                    table_hbm.at[idx_v[nbuf]], rows_v[nbuf], gsem[nbuf])
            gcp[buf].wait()
            scp[buf] = pltpu.async_copy(
                rows_v[buf], out_hbm.at[pl.ds(b0, C), f], ssem[buf])
        for b in range(2):
            if scp[b] is not None:
                scp[b].wait()

    return gather_kernel


def kernel(global_indices, table):
    B, F = global_indices.shape
    V, D = table.shape
    tt = table.T
    scratch = _build_transpose(V, D)(tt)
    tlin = scratch.reshape(-1, D)
    idxT = global_indices.T.astype(jnp.int32)
    return _build_gather(B, F, tlin.shape[0], D)(idxT, tlin)
